# baseline (speedup 1.0000x reference)
import functools

import jax
import jax.numpy as jnp
from jax import lax
from jax.experimental import pallas as pl
from jax.experimental.pallas import tpu as pltpu
from jax.experimental.pallas import tpu_sc as plsc

DIM = 64
LANES = 16
CHUNK = 128


def kernel(u, i, j, user_factors, item_factors, item_biases):
    B = u.shape[0]
    info = plsc.get_sparse_core_info()
    nw = info.num_cores * info.num_subcores
    bpw = B // nw
    n_chunks = bpw // CHUNK

    n_items = item_factors.shape[0]
    half_items = n_items // 2
    if_a = item_factors[:half_items]
    if_b = item_factors[half_items:]
    ib1 = item_biases.reshape(-1)

    mesh = plsc.VectorSubcoreMesh(core_axis_name="c", subcore_axis_name="s")

    @functools.partial(
        pl.kernel,
        mesh=mesh,
        out_type=jax.ShapeDtypeStruct((B,), jnp.float32),
        compiler_params=pltpu.CompilerParams(use_tc_tiling_on_sc=False),
        scratch_types=[
            pltpu.VMEM((bpw,), jnp.int32),
            pltpu.VMEM((bpw,), jnp.int32),
            pltpu.VMEM((bpw,), jnp.int32),
            pltpu.VMEM((bpw,), jnp.int32),
            pltpu.VMEM((bpw,), jnp.int32),
            pltpu.VMEM((bpw,), jnp.int32),
            pltpu.VMEM((bpw,), jnp.int32),
            pltpu.VMEM((CHUNK, DIM), jnp.float32),
            pltpu.VMEM((CHUNK, DIM), jnp.float32),
            pltpu.VMEM((CHUNK, DIM), jnp.float32),
            pltpu.VMEM((CHUNK, DIM), jnp.float32),
            pltpu.VMEM((CHUNK, DIM), jnp.float32),
            pltpu.VMEM((bpw,), jnp.float32),
            pltpu.VMEM((bpw,), jnp.float32),
            pltpu.VMEM((bpw,), jnp.float32),
            pltpu.SemaphoreType.DMA,
        ],
    )
    def sc_kernel(u_hbm, i_hbm, j_hbm, uf_hbm, ifa_hbm, ifb_hbm, ib_hbm,
                  out_hbm,
                  u_idx, i_idx, j_idx, i_ra, i_rb, j_ra, j_rb,
                  u_rows, ia_rows, ib_rows, ja_rows, jb_rows,
                  bi_v, bj_v, out_v, sem):
        wid = lax.axis_index("s") * info.num_cores + lax.axis_index("c")
        base = wid * bpw

        pltpu.sync_copy(u_hbm.at[pl.ds(base, bpw)], u_idx)
        pltpu.sync_copy(i_hbm.at[pl.ds(base, bpw)], i_idx)
        pltpu.sync_copy(j_hbm.at[pl.ds(base, bpw)], j_idx)

        maxa = jnp.full((LANES,), half_items - 1, jnp.int32)
        halfv = jnp.full((LANES,), half_items, jnp.int32)

        def reflect_body(g, carry):
            sl = pl.ds(g * LANES, LANES)
            iv = i_idx[sl]
            jv = j_idx[sl]
            i_ra[sl] = maxa - jnp.minimum(jnp.abs(iv - maxa), maxa)
            i_rb[sl] = jnp.minimum(jnp.abs(iv - halfv), maxa)
            j_ra[sl] = maxa - jnp.minimum(jnp.abs(jv - maxa), maxa)
            j_rb[sl] = jnp.minimum(jnp.abs(jv - halfv), maxa)
            return carry

        lax.fori_loop(0, bpw // LANES, reflect_body, 0)

        bias_copies = []
        for c in range(n_chunks):
            sl = pl.ds(c * CHUNK, CHUNK)
            bias_copies.append(pltpu.async_copy(
                ib_hbm.at[i_idx.at[sl]], bi_v.at[sl], sem))
            bias_copies.append(pltpu.async_copy(
                ib_hbm.at[j_idx.at[sl]], bj_v.at[sl], sem))
        for cp in bias_copies:
            cp.wait()

        lane_iota = lax.iota(jnp.int32, LANES)
        perms = [jnp.bitwise_xor(lane_iota, jnp.full((LANES,), s, jnp.int32))
                 for s in (1, 2, 4, 8)]
        one16 = jnp.full((LANES,), 1, jnp.int32)
        zero16 = jnp.zeros((LANES,), jnp.int32)

        def chunk_body(c, carry):
            sl = pl.ds(c * CHUNK, CHUNK)
            copies = [
                pltpu.async_copy(uf_hbm.at[u_idx.at[sl]], u_rows, sem),
                pltpu.async_copy(ifa_hbm.at[i_ra.at[sl]], ia_rows, sem),
                pltpu.async_copy(ifb_hbm.at[i_rb.at[sl]], ib_rows, sem),
                pltpu.async_copy(ifa_hbm.at[j_ra.at[sl]], ja_rows, sem),
                pltpu.async_copy(ifb_hbm.at[j_rb.at[sl]], jb_rows, sem),
            ]
            for cp in copies:
                cp.wait()

            def group_body(gg, carry2):
                gb = c * CHUNK + gg * LANES
                gsl = pl.ds(gb, LANES)
                ivec = i_idx[gsl]
                jvec = j_idx[gsl]
                hi_half = jnp.minimum(
                    jnp.maximum(ivec - (halfv - one16), zero16),
                    one16).astype(jnp.float32)
                hj_half = jnp.minimum(
                    jnp.maximum(jvec - (halfv - one16), zero16),
                    one16).astype(jnp.float32)
                acc = bi_v[gsl] - bj_v[gsl]
                for ee in range(LANES):
                    e = gg * LANES + ee
                    lane = jnp.full((LANES,), ee, jnp.int32)
                    hi = jnp.take(hi_half, lane)
                    hj = jnp.take(hj_half, lane)
                    p = None
                    for k in range(DIM // LANES):
                        ksl = pl.ds(k * LANES, LANES)
                        uv = u_rows[e, ksl]
                        ia = ia_rows[e, ksl]
                        ibv = ib_rows[e, ksl]
                        iv = ia + hi * (ibv - ia)
                        ja = ja_rows[e, ksl]
                        jb = jb_rows[e, ksl]
                        jv = ja + hj * (jb - ja)
                        t = uv * (iv - jv)
                        p = t if p is None else p + t
                    for perm in perms:
                        p = p + jnp.take(p, perm)
                    acc = jnp.where(lane_iota == ee, p + acc, acc)
                out_v[pl.ds(gb, LANES)] = acc
                return carry2

            lax.fori_loop(0, CHUNK // LANES, group_body, 0)
            return carry

        lax.fori_loop(0, n_chunks, chunk_body, 0)

        pltpu.sync_copy(out_v, out_hbm.at[pl.ds(base, bpw)])

    return sc_kernel(u, i, j, user_factors, if_a, if_b, ib1)


# speedup vs baseline: 1.2825x; 1.2825x over previous
import functools

import jax
import jax.numpy as jnp
from jax import lax
from jax.experimental import pallas as pl
from jax.experimental.pallas import tpu as pltpu
from jax.experimental.pallas import tpu_sc as plsc

DIM = 64
LANES = 16
CEX = 64


def kernel(u, i, j, user_factors, item_factors, item_biases):
    B = u.shape[0]
    info = plsc.get_sparse_core_info()
    nw = info.num_cores * info.num_subcores
    bpw = B // nw
    n_chunks = bpw // CEX
    celems = CEX * DIM

    uf1 = user_factors.reshape(-1)
    if1 = item_factors.reshape(-1)
    ib1 = item_biases.reshape(-1)

    mesh = plsc.VectorSubcoreMesh(core_axis_name="c", subcore_axis_name="s")

    @functools.partial(
        pl.kernel,
        mesh=mesh,
        out_type=jax.ShapeDtypeStruct((B,), jnp.float32),
        scratch_types=[
            pltpu.VMEM((bpw,), jnp.int32),
            pltpu.VMEM((bpw,), jnp.int32),
            pltpu.VMEM((bpw,), jnp.int32),
            pltpu.VMEM((celems,), jnp.int32),
            pltpu.VMEM((celems,), jnp.int32),
            pltpu.VMEM((celems,), jnp.int32),
            pltpu.VMEM((celems,), jnp.float32),
            pltpu.VMEM((celems,), jnp.float32),
            pltpu.VMEM((celems,), jnp.float32),
            pltpu.VMEM((bpw,), jnp.float32),
            pltpu.VMEM((bpw,), jnp.float32),
            pltpu.VMEM((bpw,), jnp.float32),
            pltpu.SemaphoreType.DMA,
        ],
    )
    def sc_kernel(u_hbm, i_hbm, j_hbm, uf_hbm, if_hbm, ib_hbm, out_hbm,
                  u_idx, i_idx, j_idx, u_ei, i_ei, j_ei,
                  u_el, i_el, j_el, bi_v, bj_v, out_v, sem):
        wid = lax.axis_index("s") * info.num_cores + lax.axis_index("c")
        base = wid * bpw

        pltpu.sync_copy(u_hbm.at[pl.ds(base, bpw)], u_idx)
        pltpu.sync_copy(i_hbm.at[pl.ds(base, bpw)], i_idx)
        pltpu.sync_copy(j_hbm.at[pl.ds(base, bpw)], j_idx)

        bias_copies = []
        for c in range(bpw // 128):
            sl = pl.ds(c * 128, 128)
            bias_copies.append(pltpu.async_copy(
                ib_hbm.at[i_idx.at[sl]], bi_v.at[sl], sem))
            bias_copies.append(pltpu.async_copy(
                ib_hbm.at[j_idx.at[sl]], bj_v.at[sl], sem))
        for cp in bias_copies:
            cp.wait()

        ngroups = CEX // LANES

        def build_lists(c):
            cb = c * CEX
            for idx_ref, ei_ref in ((u_idx, u_ei), (i_idx, i_ei),
                                    (j_idx, j_ei)):
                bases = [lax.shift_left(idx_ref[pl.ds(cb + g * LANES, LANES)],
                                        6)
                         for g in range(ngroups)]
                for d in range(DIM):
                    for g in range(ngroups):
                        ei_ref[pl.ds(d * CEX + g * LANES, LANES)] = bases[g] + d

        def compute_chunk(c):
            cb = c * CEX
            for g in range(ngroups):
                acc = (bi_v[pl.ds(cb + g * LANES, LANES)]
                       - bj_v[pl.ds(cb + g * LANES, LANES)])
                for d in range(DIM):
                    o = d * CEX + g * LANES
                    sl = pl.ds(o, LANES)
                    acc = acc + u_el[sl] * (i_el[sl] - j_el[sl])
                out_v[pl.ds(cb + g * LANES, LANES)] = acc

        def chunk_body(c, carry):
            build_lists(c)
            cp_u = pltpu.async_copy(uf_hbm.at[u_ei], u_el, sem)
            cp_i = pltpu.async_copy(if_hbm.at[i_ei], i_el, sem)
            cp_j = pltpu.async_copy(if_hbm.at[j_ei], j_el, sem)
            cp_u.wait()
            cp_i.wait()
            cp_j.wait()
            compute_chunk(c)
            return carry

        lax.fori_loop(0, n_chunks, chunk_body, 0)

        pltpu.sync_copy(out_v, out_hbm.at[pl.ds(base, bpw)])

    return sc_kernel(u, i, j, uf1, if1, ib1)


# speedup vs baseline: 2.1846x; 1.7034x over previous
import functools

import jax
import jax.numpy as jnp
from jax import lax
from jax.experimental import pallas as pl
from jax.experimental.pallas import tpu as pltpu
from jax.experimental.pallas import tpu_sc as plsc

DIM = 64
LANES = 16
NSEM = 8


def kernel(u, i, j, user_factors, item_factors, item_biases):
    B = u.shape[0]
    info = plsc.get_sparse_core_info()
    nw = info.num_cores * info.num_subcores
    bpw = B // nw

    uf2 = user_factors.reshape(-1, 2 * DIM)
    ib1 = item_biases.reshape(-1)

    mesh = plsc.VectorSubcoreMesh(core_axis_name="c", subcore_axis_name="s")
    half = bpw // 2

    @functools.partial(
        pl.kernel,
        mesh=mesh,
        out_type=jax.ShapeDtypeStruct((B,), jnp.float32),
        scratch_types=[
            pltpu.VMEM((bpw,), jnp.int32),
            pltpu.VMEM((bpw,), jnp.int32),
            pltpu.VMEM((bpw,), jnp.int32),
            pltpu.VMEM((bpw,), jnp.int32),
            pltpu.VMEM((half, 2 * DIM), jnp.float32),
            pltpu.VMEM((half, DIM), jnp.float32),
            pltpu.VMEM((half, DIM), jnp.float32),
            pltpu.VMEM((bpw,), jnp.float32),
            pltpu.VMEM((bpw,), jnp.float32),
            pltpu.VMEM((bpw,), jnp.float32),
            [pltpu.SemaphoreType.DMA] * NSEM,
        ],
    )
    def sc_kernel(u_hbm, i_hbm, j_hbm, uf_hbm, if_hbm, ib_hbm, out_hbm,
                  u_idx, i_idx, j_idx, u_sh,
                  u_rows, i_rows, j_rows, bi_v, bj_v, out_v, sems):
        wid = lax.axis_index("s") * info.num_cores + lax.axis_index("c")
        base = wid * bpw

        pltpu.sync_copy(u_hbm.at[pl.ds(base, bpw)], u_idx)
        pltpu.sync_copy(i_hbm.at[pl.ds(base, bpw)], i_idx)
        pltpu.sync_copy(j_hbm.at[pl.ds(base, bpw)], j_idx)

        def shift_body(g, carry):
            sl = pl.ds(g * LANES, LANES)
            u_sh[sl] = lax.shift_right_logical(u_idx[sl], 1)
            return carry

        lax.fori_loop(0, bpw // LANES, shift_body, 0)

        bias_copies = [
            pltpu.async_copy(ib_hbm.at[i_idx], bi_v, sems[0]),
            pltpu.async_copy(ib_hbm.at[j_idx], bj_v, sems[1]),
        ]
        for cp in bias_copies:
            cp.wait()

        lane_iota = lax.iota(jnp.int32, LANES)
        perms = [jnp.bitwise_xor(lane_iota, jnp.full((LANES,), s, jnp.int32))
                 for s in (1, 2, 4, 8)]
        one16 = jnp.full((LANES,), 1, jnp.int32)

        rows_per_sem = half * 2 // NSEM
        for h in range(2):
            hb = h * half

            cp_u = pltpu.async_copy(
                uf_hbm.at[u_sh.at[pl.ds(hb, half)]], u_rows, sems[0])

            def fetch_body(g, carry):
                gb = g * LANES
                ivec = i_idx[pl.ds(hb + gb, LANES)]
                jvec = j_idx[pl.ds(hb + gb, LANES)]
                for ee in range(LANES):
                    e = gb + ee
                    s0 = (ee * 2) % NSEM
                    pltpu.async_copy(if_hbm.at[ivec[ee]], i_rows.at[e],
                                     sems[s0])
                    pltpu.async_copy(if_hbm.at[jvec[ee]], j_rows.at[e],
                                     sems[s0 + 1])
                return carry

            lax.fori_loop(0, half // LANES, fetch_body, 0)

            cp_u.wait()
            for s in range(NSEM):
                pltpu.make_async_copy(
                    if_hbm.at[pl.ds(0, rows_per_sem)],
                    i_rows.at[pl.ds(0, rows_per_sem)],
                    sems[s]).wait()

            def group_body(gg, carry):
                gb = gg * LANES
                pu = jnp.bitwise_and(u_idx[pl.ds(hb + gb, LANES)],
                                     one16).astype(jnp.float32)
                acc = (bi_v[pl.ds(hb + gb, LANES)]
                       - bj_v[pl.ds(hb + gb, LANES)])
                for ee in range(LANES):
                    e = gb + ee
                    lane = jnp.full((LANES,), ee, jnp.int32)
                    fu = jnp.take(pu, lane)
                    p = None
                    for k in range(DIM // LANES):
                        ksl = pl.ds(k * LANES, LANES)
                        ul = u_rows[e, ksl]
                        uh = u_rows[e, pl.ds(DIM + k * LANES, LANES)]
                        uv = ul + fu * (uh - ul)
                        t = uv * (i_rows[e, ksl] - j_rows[e, ksl])
                        p = t if p is None else p + t
                    for perm in perms:
                        p = p + jnp.take(p, perm)
                    acc = jnp.where(lane_iota == ee, p + acc, acc)
                out_v[pl.ds(hb + gb, LANES)] = acc
                return carry

            lax.fori_loop(0, half // LANES, group_body, 0)

        pltpu.sync_copy(out_v, out_hbm.at[pl.ds(base, bpw)])

    return sc_kernel(u, i, j, uf2, item_factors, ib1)


# speedup vs baseline: 2.2683x; 1.0383x over previous
import functools

import jax
import jax.numpy as jnp
from jax import lax
from jax.experimental import pallas as pl
from jax.experimental.pallas import tpu as pltpu
from jax.experimental.pallas import tpu_sc as plsc

DIM = 64
LANES = 16
NSEM = 8


def kernel(u, i, j, user_factors, item_factors, item_biases):
    B = u.shape[0]
    info = plsc.get_sparse_core_info()
    nw = info.num_cores * info.num_subcores
    bpw = B // nw

    ib1 = item_biases.reshape(-1)

    mesh = plsc.VectorSubcoreMesh(core_axis_name="c", subcore_axis_name="s")

    @functools.partial(
        pl.kernel,
        mesh=mesh,
        out_type=jax.ShapeDtypeStruct((B,), jnp.float32),
        scratch_types=[
            pltpu.VMEM((bpw,), jnp.int32),
            pltpu.VMEM((bpw,), jnp.int32),
            pltpu.VMEM((bpw,), jnp.int32),
            pltpu.VMEM((bpw // 2, DIM), jnp.float32),
            pltpu.VMEM((bpw // 2, DIM), jnp.float32),
            pltpu.VMEM((bpw // 2, DIM), jnp.float32),
            pltpu.VMEM((bpw,), jnp.float32),
            pltpu.VMEM((bpw,), jnp.float32),
            pltpu.VMEM((bpw,), jnp.float32),
            [pltpu.SemaphoreType.DMA] * NSEM,
        ],
    )
    def sc_kernel(u_hbm, i_hbm, j_hbm, uf_hbm, if_hbm, ib_hbm, out_hbm,
                  u_idx, i_idx, j_idx,
                  u_rows, i_rows, j_rows, bi_v, bj_v, out_v, sems):
        wid = lax.axis_index("s") * info.num_cores + lax.axis_index("c")
        base = wid * bpw

        pltpu.sync_copy(u_hbm.at[pl.ds(base, bpw)], u_idx)
        pltpu.sync_copy(i_hbm.at[pl.ds(base, bpw)], i_idx)
        pltpu.sync_copy(j_hbm.at[pl.ds(base, bpw)], j_idx)

        bias_copies = []
        for c in range(bpw // 128):
            sl = pl.ds(c * 128, 128)
            bias_copies.append(pltpu.async_copy(
                ib_hbm.at[i_idx.at[sl]], bi_v.at[sl], sems[0]))
            bias_copies.append(pltpu.async_copy(
                ib_hbm.at[j_idx.at[sl]], bj_v.at[sl], sems[1]))
        for cp in bias_copies:
            cp.wait()

        lane_iota = lax.iota(jnp.int32, LANES)
        perms = [jnp.bitwise_xor(lane_iota, jnp.full((LANES,), s, jnp.int32))
                 for s in (1, 2, 4, 8)]

        half = bpw // 2
        rows_per_sem = half * 3 // NSEM
        for h in range(2):
            hb = h * half

            def fetch_body(g, carry):
                gb = g * LANES
                uvec = u_idx[pl.ds(hb + gb, LANES)]
                ivec = i_idx[pl.ds(hb + gb, LANES)]
                jvec = j_idx[pl.ds(hb + gb, LANES)]
                for ee in range(LANES):
                    e = gb + ee
                    s0 = (ee * 3) % NSEM
                    pltpu.async_copy(uf_hbm.at[uvec[ee]], u_rows.at[e],
                                     sems[s0])
                    pltpu.async_copy(if_hbm.at[ivec[ee]], i_rows.at[e],
                                     sems[(s0 + 1) % NSEM])
                    pltpu.async_copy(if_hbm.at[jvec[ee]], j_rows.at[e],
                                     sems[(s0 + 2) % NSEM])
                return carry

            lax.fori_loop(0, half // LANES, fetch_body, 0)

            for s in range(NSEM):
                pltpu.make_async_copy(
                    uf_hbm.at[pl.ds(0, rows_per_sem)],
                    u_rows.at[pl.ds(0, rows_per_sem)],
                    sems[s]).wait()

            def group_body(gg, carry):
                gb = gg * LANES
                acc = (bi_v[pl.ds(hb + gb, LANES)]
                       - bj_v[pl.ds(hb + gb, LANES)])
                for ee in range(LANES):
                    e = gb + ee
                    p = None
                    for k in range(DIM // LANES):
                        ksl = pl.ds(k * LANES, LANES)
                        t = u_rows[e, ksl] * (i_rows[e, ksl] - j_rows[e, ksl])
                        p = t if p is None else p + t
                    for perm in perms:
                        p = p + jnp.take(p, perm)
                    acc = jnp.where(lane_iota == ee, p + acc, acc)
                out_v[pl.ds(hb + gb, LANES)] = acc
                return carry

            lax.fori_loop(0, half // LANES, group_body, 0)

        pltpu.sync_copy(out_v, out_hbm.at[pl.ds(base, bpw)])

    return sc_kernel(u, i, j, user_factors, item_factors, ib1)
